# baseline (speedup 1.0000x reference)
import functools

import jax
import jax.numpy as jnp
from jax import lax
from jax.experimental import pallas as pl
from jax.experimental.pallas import tpu as pltpu
from jax.experimental.pallas import tpu_sc as plsc
from jax._src.pallas import mpmd as _mpmd

B, T, S, V = 8, 32, 200, 100000
NROWS = B * T

NC, NS = 2, 16
NW = NC * NS
PER_W = (B * T * S) // NW
NCHUNK = 16
CHUNK = PER_W // NCHUNK

def _sc_mesh():
    return plsc.VectorSubcoreMesh(
        core_axis_name="c",
        subcore_axis_name="s",
        num_cores=NC,
        num_subcores=NS,
    )


def _worker_id():
    return lax.axis_index("s") * NC + lax.axis_index("c")


@functools.cache
def _make_sc_gather():
    @functools.partial(
        pl.kernel,
        out_type=jax.ShapeDtypeStruct((NW, NCHUNK, CHUNK), jnp.float32),
        mesh=_sc_mesh(),
        scratch_types=[
            pltpu.VMEM((NCHUNK, CHUNK), jnp.int32),
            pltpu.VMEM((NCHUNK, CHUNK), jnp.float32),
            pltpu.SemaphoreType.DMA,
        ],
    )
    def _sc_gather(x_hbm, pos_hbm, g_hbm, idx_v, val_v, sem):
        wid = _worker_id()
        pltpu.sync_copy(pos_hbm.at[wid], idx_v)
        copies = [
            pltpu.async_copy(x_hbm.at[idx_v.at[j]], val_v.at[j], sem)
            for j in range(NCHUNK)
        ]
        for c in copies:
            c.wait()
        pltpu.sync_copy(val_v, g_hbm.at[wid])

    return _sc_gather


@functools.cache
def _make_sc_scatter():
    def _sc_scatter_body(y_in, pos_hbm, new_hbm, y_out, idx_v, val_v, sem):
        del y_in
        wid = _worker_id()
        pltpu.sync_copy(pos_hbm.at[wid], idx_v)
        pltpu.sync_copy(new_hbm.at[wid], val_v)
        copies = [
            pltpu.async_copy(val_v.at[j], y_out.at[idx_v.at[j]], sem)
            for j in range(NCHUNK)
        ]
        for c in copies:
            c.wait()

    return _mpmd._mpmd_map(
        [(_sc_mesh(), _sc_scatter_body)],
        out_types=[jax.ShapeDtypeStruct((NROWS * V,), jnp.float32)],
        input_output_aliases={0: 0},
        scratch_types=[
            pltpu.VMEM((NCHUNK, CHUNK), jnp.int32),
            pltpu.VMEM((NCHUNK, CHUNK), jnp.float32),
            pltpu.SemaphoreType.DMA,
        ],
    )


ROWS_PER_BLK = 16


def _tc_main_body(x_ref, w_ref, b_ref, y_ref, st_ref):
    x = x_ref[...]
    w = w_ref[...]
    m = jnp.max(x, axis=1, keepdims=True)
    e = jnp.exp(x - m)
    sumexp = jnp.sum(e, axis=1, keepdims=True)
    dot = jnp.sum(x * w, axis=1, keepdims=True)
    p = jax.nn.sigmoid(dot + b_ref[0, 0])
    base = (1.0 - p) * (e / sumexp) + 1e-10
    y_ref[...] = jnp.log(base)
    zeros = jnp.zeros((ROWS_PER_BLK, 5), jnp.float32)
    st_ref[...] = jnp.concatenate([m, sumexp, dot, zeros], axis=1)


def _tc_main(x2, w2, b2):
    return pl.pallas_call(
        _tc_main_body,
        grid=(NROWS // ROWS_PER_BLK,),
        in_specs=[
            pl.BlockSpec((ROWS_PER_BLK, V), lambda i: (i, 0)),
            pl.BlockSpec((1, V), lambda i: (0, 0)),
            pl.BlockSpec((1, 1), lambda i: (0, 0)),
        ],
        out_specs=[
            pl.BlockSpec((ROWS_PER_BLK, V), lambda i: (i, 0)),
            pl.BlockSpec((ROWS_PER_BLK, 8), lambda i: (i, 0)),
        ],
        out_shape=[
            jax.ShapeDtypeStruct((NROWS, V), jnp.float32),
            jax.ShapeDtypeStruct((NROWS, 8), jnp.float32),
        ],
    )(x2, w2, b2)


def _tc_fixup_body(g_ref, st_ref, att_ref, src_ref, b_ref, new_ref):
    m = st_ref[:, :, 0:1]
    sumexp = st_ref[:, :, 1:2]
    dot = st_ref[:, :, 2:3]
    p = jax.nn.sigmoid(dot + b_ref[0, 0])
    lane = lax.broadcasted_iota(jnp.int32, (B, T, S), 2)
    copy = jnp.where(lane < S, p * att_ref[...], 0.0)
    src = src_ref[...]
    eq = (src[:, :, None] == src[:, None, :]).astype(jnp.float32)
    totals = []
    for bb in range(B):
        totals.append(
            jnp.dot(copy[bb], eq[bb], preferred_element_type=jnp.float32)
        )
    tot = jnp.stack(totals)
    gen = (1.0 - p) * (jnp.exp(g_ref[...] - m) / sumexp)
    new_ref[...] = jnp.log(gen + tot + 1e-10)


def _tc_fixup(g, st, att, src, b2):
    return pl.pallas_call(
        _tc_fixup_body,
        out_shape=jax.ShapeDtypeStruct((B, T, S), jnp.float32),
    )(g, st, att, src, b2)


def kernel(src, output, attention, W, b):
    src = src.astype(jnp.int32)
    row = jnp.arange(NROWS, dtype=jnp.int32)[:, None] * V
    pos = row + jnp.broadcast_to(src[:, None, :], (B, T, S)).reshape(NROWS, S)
    pos3 = pos.reshape(NW, NCHUNK, CHUNK)

    xflat = output.reshape(-1)
    g3 = _make_sc_gather()(xflat, pos3)

    x2 = output.reshape(NROWS, V)
    w2 = W.reshape(1, V)
    b2 = b.reshape(1, 1)
    y, st = _tc_main(x2, w2, b2)

    g = g3.reshape(B, T, S)
    st_b = st.reshape(B, T, 8)
    newvals = _tc_fixup(g, st_b, attention, src, b2)

    (yflat,) = _make_sc_scatter()(
        y.reshape(-1), pos3, newvals.reshape(NW, NCHUNK, CHUNK)
    )
    return (yflat.reshape(B, T, V), attention)


# speedup vs baseline: 4.6450x; 4.6450x over previous
import jax
import jax.numpy as jnp
from jax import lax
from jax.experimental import pallas as pl
from jax.experimental.pallas import tpu as pltpu

B, T, S, V = 8, 32, 200, 100000
NROWS = B * T
R = 16
SP = 256


def _tc_body(x_ref, w_ref, att_ref, src_ref, src_sm, b_ref, y_ref, g_sc, val_sc):
    bidx = pl.program_id(0) // (T // R)
    x = x_ref[...]
    w = w_ref[...]
    m = jnp.max(x, axis=1, keepdims=True)
    e = jnp.exp(x - m)
    sumexp = jnp.sum(e, axis=1, keepdims=True)
    dot = jnp.sum(x * w, axis=1, keepdims=True)
    p = jax.nn.sigmoid(dot + b_ref[0, 0])
    y_ref[...] = jnp.log((1.0 - p) * (e / sumexp) + 1e-10)

    lane128 = lax.broadcasted_iota(jnp.int32, (1, 128), 1)
    for s in range(S):
        col = src_sm[bidx, s]
        hi = pl.multiple_of((col // 128) * 128, 128)
        xs = x_ref[:, pl.ds(hi, 128)]
        sel = lane128 == (col - hi)
        g_sc[:, s : s + 1] = jnp.sum(
            jnp.where(sel, xs, 0.0), axis=1, keepdims=True
        )

    lane = lax.broadcasted_iota(jnp.int32, (1, SP), 1)
    att = jnp.where(lane < S, att_ref[...], 0.0)
    copy = p * att
    src = jnp.where(lane < S, src_ref[0], -1)
    eq = (src[0, :, None] == src[0, None, :]).astype(jnp.float32)
    tot = jnp.dot(copy, eq, preferred_element_type=jnp.float32)

    g = g_sc[...]
    val_sc[...] = jnp.log((1.0 - p) * (jnp.exp(g - m) / sumexp) + 1e-10 + tot)

    for s in range(S):
        col = src_sm[bidx, s]
        hi = pl.multiple_of((col // 128) * 128, 128)
        ys = y_ref[:, pl.ds(hi, 128)]
        sel = lane128 == (col - hi)
        y_ref[:, pl.ds(hi, 128)] = jnp.where(sel, val_sc[:, s : s + 1], ys)


def _tc_fused(x2, w2, att2, src2, b2):
    blocks_per_batch = T // R
    return pl.pallas_call(
        _tc_body,
        grid=(NROWS // R,),
        in_specs=[
            pl.BlockSpec((R, V), lambda i: (i, 0)),
            pl.BlockSpec((1, V), lambda i: (0, 0)),
            pl.BlockSpec((R, SP), lambda i: (i, 0)),
            pl.BlockSpec((1, 1, SP), lambda i: (i // blocks_per_batch, 0, 0)),
            pl.BlockSpec(memory_space=pltpu.SMEM),
            pl.BlockSpec((1, 1), lambda i: (0, 0)),
        ],
        out_specs=pl.BlockSpec((R, V), lambda i: (i, 0)),
        out_shape=jax.ShapeDtypeStruct((NROWS, V), jnp.float32),
        scratch_shapes=[
            pltpu.VMEM((R, SP), jnp.float32),
            pltpu.VMEM((R, SP), jnp.float32),
        ],
    )(x2, w2, att2, src2.reshape(B, 1, SP), src2, b2)


def kernel(src, output, attention, W, b):
    src = src.astype(jnp.int32)
    x2 = output.reshape(NROWS, V)
    w2 = W.reshape(1, V)
    b2 = b.reshape(1, 1)
    att2 = jnp.pad(attention.reshape(NROWS, S), ((0, 0), (0, SP - S)))
    src2 = jnp.pad(src, ((0, 0), (0, SP - S)))
    y = _tc_fused(x2, w2, att2, src2, b2)
    return (y.reshape(B, T, V), attention)


# speedup vs baseline: 5.0219x; 1.0811x over previous
import jax
import jax.numpy as jnp
from jax import lax
from jax.experimental import pallas as pl
from jax.experimental.pallas import tpu as pltpu

B, T, S, V = 8, 32, 200, 100000
NROWS = B * T
R = 16
SP = 256


def _tc_body(x_ref, w_ref, att_ref, src_ref, src_sm, b_ref, y_ref, g_sc, val_sc):
    bidx = pl.program_id(0) // (T // R)
    x = x_ref[...]
    w = w_ref[...]
    m = jnp.max(x, axis=1, keepdims=True)
    e = jnp.exp(x - m)
    sumexp = jnp.sum(e, axis=1, keepdims=True)
    dot = jnp.sum(x * w, axis=1, keepdims=True)
    p = jax.nn.sigmoid(dot + b_ref[0, 0])
    scale = (1.0 - p) / sumexp
    y_ref[...] = jnp.log(e * scale + 1e-10)

    lane128 = lax.broadcasted_iota(jnp.int32, (1, 128), 1)
    for s in range(S):
        col = src_sm[bidx, s]
        hi = pl.multiple_of((col // 128) * 128, 128)
        xs = x_ref[:, pl.ds(hi, 128)]
        sel = lane128 == (col - hi)
        g_sc[:, s : s + 1] = jnp.sum(
            jnp.where(sel, xs, 0.0), axis=1, keepdims=True
        )

    lane = lax.broadcasted_iota(jnp.int32, (1, SP), 1)
    att = jnp.where(lane < S, att_ref[...], 0.0)
    copy = p * att
    src = jnp.where(lane < S, src_ref[0], -1)
    eq = (src[0, :, None] == src[0, None, :]).astype(jnp.float32)
    tot = jnp.dot(copy, eq, preferred_element_type=jnp.float32)

    g = g_sc[...]
    val_sc[...] = jnp.log(jnp.exp(g - m) * scale + 1e-10 + tot)

    for s in range(S):
        col = src_sm[bidx, s]
        hi = pl.multiple_of((col // 128) * 128, 128)
        ys = y_ref[:, pl.ds(hi, 128)]
        sel = lane128 == (col - hi)
        y_ref[:, pl.ds(hi, 128)] = jnp.where(sel, val_sc[:, s : s + 1], ys)


def _tc_fused(x2, w2, att2, src2, b2):
    blocks_per_batch = T // R
    return pl.pallas_call(
        _tc_body,
        grid=(NROWS // R,),
        in_specs=[
            pl.BlockSpec((R, V), lambda i: (i, 0)),
            pl.BlockSpec((1, V), lambda i: (0, 0)),
            pl.BlockSpec((R, SP), lambda i: (i, 0)),
            pl.BlockSpec((1, 1, SP), lambda i: (i // blocks_per_batch, 0, 0)),
            pl.BlockSpec(memory_space=pltpu.SMEM),
            pl.BlockSpec((1, 1), lambda i: (0, 0)),
        ],
        out_specs=pl.BlockSpec((R, V), lambda i: (i, 0)),
        out_shape=jax.ShapeDtypeStruct((NROWS, V), jnp.float32),
        scratch_shapes=[
            pltpu.VMEM((R, SP), jnp.float32),
            pltpu.VMEM((R, SP), jnp.float32),
        ],
    )(x2, w2, att2, src2.reshape(B, 1, SP), src2, b2)


def kernel(src, output, attention, W, b):
    src = src.astype(jnp.int32)
    x2 = output.reshape(NROWS, V)
    w2 = W.reshape(1, V)
    b2 = b.reshape(1, 1)
    att2 = jnp.pad(attention.reshape(NROWS, S), ((0, 0), (0, SP - S)))
    src2 = jnp.pad(src, ((0, 0), (0, SP - S)))
    y = _tc_fused(x2, w2, att2, src2, b2)
    return (y.reshape(B, T, V), attention)
